# baseline (speedup 1.0000x reference)
import functools

import jax
import jax.numpy as jnp
from jax import lax
from jax.experimental import pallas as pl
from jax.experimental.pallas import tpu as pltpu
from jax.experimental.pallas import tpu_sc as plsc

_MAX_VAL = 2097151
_NC = 2
_NS = 16
_NW = _NC * _NS
_L = 16
_NQ = 4
_NSUB = 8
_NPH = 4


@functools.lru_cache(maxsize=None)
def _build(n):
    assert n % (_NW * _NQ * 4 * _L * _NSUB) == 0, n
    e_tile = n // _NW
    e_sub = e_tile // _NSUB
    w_sub = e_sub // 4
    e_q = e_tile // _NQ
    w_q = e_q // 4
    e_ph = e_q // _NPH
    w_ph = e_ph // 4

    mesh = plsc.VectorSubcoreMesh(
        core_axis_name="c", subcore_axis_name="s",
        num_cores=_NC, num_subcores=_NS,
    )
    cparams = pltpu.CompilerParams(needs_layout_passes=False)

    @functools.partial(
        pl.kernel,
        out_type=(
            jax.ShapeDtypeStruct((n // 4,), jnp.int32),
            jax.ShapeDtypeStruct((_NW * _NQ, _L), jnp.int32),
        ),
        mesh=mesh,
        scratch_types=[
            pltpu.VMEM((2, e_sub), jnp.int32),
            pltpu.VMEM((2, w_sub), jnp.int32),
            pltpu.VMEM((_NQ, _L), jnp.int32),
            pltpu.SemaphoreType.DMA,
            pltpu.SemaphoreType.DMA,
            pltpu.SemaphoreType.DMA,
            pltpu.SemaphoreType.DMA,
        ],
        compiler_params=cparams,
    )
    def _pack_kernel(mask_hbm, pw_hbm, sums_hbm, mbuf, pbuf, qsums,
                     isem0, isem1, osem0, osem1):
        wid = lax.axis_index("c") * _NS + lax.axis_index("s")
        base_e = wid * e_tile
        base_w = wid * (e_tile // 4)
        isems = (isem0, isem1)
        osems = (osem0, osem1)

        def start_in(sub):
            cur = sub % 2
            return pltpu.async_copy(
                mask_hbm.at[pl.ds(base_e + sub * e_sub, e_sub)],
                mbuf.at[cur], isems[cur])

        in_h = {0: start_in(0)}
        out_h = {}
        acc32 = jnp.zeros((_L,), jnp.int32)
        for sub in range(_NSUB):
            cur = sub % 2
            if sub + 1 < _NSUB:
                in_h[sub + 1] = start_in(sub + 1)
            in_h.pop(sub).wait()
            if sub - 2 in out_h:
                out_h.pop(sub - 2).wait()

            n_it = w_sub // _L
            half = n_it // 2
            for blk in range(2):

                def it(j, accb, blk=blk, cur=cur):
                    jj = blk * half + j
                    v0 = mbuf[cur, pl.ds(jj * 4 * _L, _L)]
                    v1 = mbuf[cur, pl.ds((jj * 4 + 1) * _L, _L)]
                    v2 = mbuf[cur, pl.ds((jj * 4 + 2) * _L, _L)]
                    v3 = mbuf[cur, pl.ds((jj * 4 + 3) * _L, _L)]
                    packed = (v0 + (v1 << 8)) + ((v2 << 16) + (v3 << 24))
                    pbuf[cur, pl.ds(jj * _L, _L)] = packed
                    return accb + packed

                accb = lax.fori_loop(0, half, it, jnp.zeros((_L,), jnp.int32))
                acc32 = acc32 + (
                    (accb & 0xFF) + ((accb >> 8) & 0xFF)
                    + ((accb >> 16) & 0xFF) + ((accb >> 24) & 0xFF))

            out_h[sub] = pltpu.async_copy(
                pbuf.at[cur], pw_hbm.at[pl.ds(base_w + sub * w_sub, w_sub)],
                osems[cur])
            if sub % 2 == 1:
                qsums[sub // 2] = acc32
                acc32 = jnp.zeros((_L,), jnp.int32)
        for h in out_h.values():
            h.wait()
        pltpu.sync_copy(qsums, sums_hbm.at[pl.ds(wid * _NQ, _NQ)])

    @functools.partial(
        pl.kernel,
        out_type=jax.ShapeDtypeStruct((n,), jnp.int32),
        mesh=mesh,
        scratch_types=[
            pltpu.VMEM((2, _NQ, w_ph), jnp.int32),
            pltpu.VMEM((2, _NQ, e_ph), jnp.int32),
            pltpu.VMEM((_NW * _NQ, _L), jnp.int32),
            pltpu.SemaphoreType.DMA,
            pltpu.SemaphoreType.DMA,
            pltpu.SemaphoreType.DMA,
            pltpu.SemaphoreType.DMA,
        ],
        compiler_params=cparams,
    )
    def _scan_kernel(pw_hbm, sums_hbm, out_hbm, pbuf, obuf, sums_v,
                     isem0, isem1, osem0, osem1):
        wid = lax.axis_index("c") * _NS + lax.axis_index("s")
        isems = (isem0, isem1)
        osems = (osem0, osem1)
        pltpu.sync_copy(sums_hbm, sums_v)

        def carries(q):
            def acc_row(r, cv):
                m = (r < wid * _NQ + q).astype(jnp.int32)
                return cv + sums_v[r] * m

            cv = lax.fori_loop(0, _NW * _NQ, acc_row,
                               jnp.zeros((_L,), jnp.int32))
            return jnp.sum(cv) - 1

        carry = [carries(q) for q in range(_NQ)]

        def start_in(ph):
            cur = ph % 2
            return [
                pltpu.async_copy(
                    pw_hbm.at[pl.ds(
                        (wid * _NQ + q) * w_q + ph * w_ph, w_ph)],
                    pbuf.at[cur, q], isems[cur])
                for q in range(_NQ)
            ]

        in_h = {0: start_in(0)}
        out_h = {}
        for ph in range(_NPH):
            cur = ph % 2
            if ph + 1 < _NPH:
                in_h[ph + 1] = start_in(ph + 1)
            for h in in_h.pop(ph):
                h.wait()
            if ph - 2 in out_h:
                for h in out_h.pop(ph - 2):
                    h.wait()

            def it(j, carry, cur=cur):
                new = []
                for q in range(_NQ):
                    pv = pbuf[cur, q, pl.ds(j * _L, _L)]
                    incl = plsc.cumsum(pv)
                    s = jnp.sum(pv)
                    cbef = s * 0x01010100
                    b0 = incl & 0xFF
                    b1 = (incl >> 8) & 0xFF
                    b2 = (incl >> 16) & 0xFF
                    b3 = incl >> 24
                    c = carry[q]
                    o0 = c + b0
                    o1 = (c + ((cbef >> 8) & 0xFF)) + b1
                    o2 = (c + ((cbef >> 16) & 0xFF)) + b2
                    o3 = (c + (cbef >> 24)) + b3
                    zero = jnp.int32(0)
                    obuf[cur, q, pl.ds(j * 4 * _L, _L)] = jnp.minimum(
                        jnp.maximum(o0, zero), _MAX_VAL)
                    obuf[cur, q, pl.ds((j * 4 + 1) * _L, _L)] = jnp.minimum(
                        jnp.maximum(o1, zero), _MAX_VAL)
                    obuf[cur, q, pl.ds((j * 4 + 2) * _L, _L)] = jnp.minimum(
                        jnp.maximum(o2, zero), _MAX_VAL)
                    obuf[cur, q, pl.ds((j * 4 + 3) * _L, _L)] = jnp.minimum(
                        jnp.maximum(o3, zero), _MAX_VAL)
                    new.append(c + ((cbef >> 24) + (s >> 24)))
                return tuple(new)

            carry = list(lax.fori_loop(0, w_ph // _L, it, tuple(carry)))

            out_h[ph] = [
                pltpu.async_copy(
                    obuf.at[cur, q],
                    out_hbm.at[pl.ds(
                        wid * e_tile + q * e_q + ph * e_ph, e_ph)],
                    osems[cur])
                for q in range(_NQ)
            ]
        for hs in out_h.values():
            for h in hs:
                h.wait()

    def run(mask_i32):
        pw, sums = _pack_kernel(mask_i32)
        return _scan_kernel(pw, sums)

    return run


def kernel(mask_1d):
    n = mask_1d.shape[0]
    return _build(n)(mask_1d.astype(jnp.int32))


# speedup vs baseline: 1.1606x; 1.1606x over previous
import functools

import jax
import jax.numpy as jnp
from jax import lax
from jax.experimental import pallas as pl
from jax.experimental.pallas import tpu as pltpu
from jax.experimental.pallas import tpu_sc as plsc

_MAX_VAL = 2097151
_NC = 2
_NS = 16
_NW = _NC * _NS
_L = 16
_NQ = 4
_NSUB = 8
_NPH = 4


@functools.lru_cache(maxsize=None)
def _build(n):
    assert n % (_NW * _NQ * 4 * _L * _NSUB) == 0, n
    e_tile = n // _NW
    e_sub = e_tile // _NSUB
    w_sub = e_sub // 4
    e_q = e_tile // _NQ
    w_q = e_q // 4
    e_ph = e_q // _NPH
    w_ph = e_ph // 4

    mesh = plsc.VectorSubcoreMesh(
        core_axis_name="c", subcore_axis_name="s",
        num_cores=_NC, num_subcores=_NS,
    )
    cparams = pltpu.CompilerParams(needs_layout_passes=False)

    @functools.partial(
        pl.kernel,
        out_type=(
            jax.ShapeDtypeStruct((n // 4,), jnp.int32),
            jax.ShapeDtypeStruct((_NW * _NQ, _L), jnp.int32),
        ),
        mesh=mesh,
        scratch_types=[
            pltpu.VMEM((2, e_sub), jnp.int32),
            pltpu.VMEM((2, w_sub), jnp.int32),
            pltpu.VMEM((_NQ, _L), jnp.int32),
            pltpu.SemaphoreType.DMA,
            pltpu.SemaphoreType.DMA,
            pltpu.SemaphoreType.DMA,
            pltpu.SemaphoreType.DMA,
        ],
        compiler_params=cparams,
    )
    def _pack_kernel(mask_hbm, pw_hbm, sums_hbm, mbuf, pbuf, qsums,
                     isem0, isem1, osem0, osem1):
        wid = lax.axis_index("c") * _NS + lax.axis_index("s")
        base_e = wid * e_tile
        base_w = wid * (e_tile // 4)
        isems = (isem0, isem1)
        osems = (osem0, osem1)

        def start_in(sub):
            cur = sub % 2
            return pltpu.async_copy(
                mask_hbm.at[pl.ds(base_e + sub * e_sub, e_sub)],
                mbuf.at[cur], isems[cur])

        in_h = {0: start_in(0)}
        out_h = {}
        acc32 = jnp.zeros((_L,), jnp.int32)
        for sub in range(_NSUB):
            cur = sub % 2
            if sub + 1 < _NSUB:
                in_h[sub + 1] = start_in(sub + 1)
            in_h.pop(sub).wait()
            if sub - 2 in out_h:
                out_h.pop(sub - 2).wait()

            n_it = w_sub // _L
            half = n_it // 2
            for blk in range(2):

                def it(j, accb, blk=blk, cur=cur):
                    acc = accb
                    for u in range(2):
                        jj = blk * half + j * 2 + u
                        v0 = mbuf[cur, pl.ds(jj * 4 * _L, _L)]
                        v1 = mbuf[cur, pl.ds((jj * 4 + 1) * _L, _L)]
                        v2 = mbuf[cur, pl.ds((jj * 4 + 2) * _L, _L)]
                        v3 = mbuf[cur, pl.ds((jj * 4 + 3) * _L, _L)]
                        packed = (v0 + (v1 << 8)) + ((v2 << 16) + (v3 << 24))
                        pbuf[cur, pl.ds(jj * _L, _L)] = packed
                        acc = acc + packed
                    return acc

                accb = lax.fori_loop(0, half // 2, it,
                                     jnp.zeros((_L,), jnp.int32))
                acc32 = acc32 + (
                    (accb & 0xFF) + ((accb >> 8) & 0xFF)
                    + ((accb >> 16) & 0xFF) + ((accb >> 24) & 0xFF))

            out_h[sub] = pltpu.async_copy(
                pbuf.at[cur], pw_hbm.at[pl.ds(base_w + sub * w_sub, w_sub)],
                osems[cur])
            if sub % 2 == 1:
                qsums[sub // 2] = acc32
                acc32 = jnp.zeros((_L,), jnp.int32)
        for h in out_h.values():
            h.wait()
        pltpu.sync_copy(qsums, sums_hbm.at[pl.ds(wid * _NQ, _NQ)])

    @functools.partial(
        pl.kernel,
        out_type=jax.ShapeDtypeStruct((n,), jnp.int32),
        mesh=mesh,
        scratch_types=[
            pltpu.VMEM((2, _NQ, w_ph), jnp.int32),
            pltpu.VMEM((2, _NQ, e_ph), jnp.int32),
            pltpu.VMEM((_NW * _NQ, _L), jnp.int32),
            pltpu.SemaphoreType.DMA,
            pltpu.SemaphoreType.DMA,
            pltpu.SemaphoreType.DMA,
            pltpu.SemaphoreType.DMA,
        ],
        compiler_params=cparams,
    )
    def _scan_kernel(pw_hbm, sums_hbm, out_hbm, pbuf, obuf, sums_v,
                     isem0, isem1, osem0, osem1):
        wid = lax.axis_index("c") * _NS + lax.axis_index("s")
        isems = (isem0, isem1)
        osems = (osem0, osem1)
        pltpu.sync_copy(sums_hbm, sums_v)

        def carries(q):
            def acc_row(r, cv):
                m = (r < wid * _NQ + q).astype(jnp.int32)
                return cv + sums_v[r] * m

            cv = lax.fori_loop(0, _NW * _NQ, acc_row,
                               jnp.zeros((_L,), jnp.int32))
            return jnp.full((_L,), jnp.sum(cv) - 1, jnp.int32)

        carry = [carries(q) for q in range(_NQ)]
        idx15 = jnp.full((_L, 1), _L - 1, jnp.int32)
        gdn = lax.GatherDimensionNumbers(
            offset_dims=(), collapsed_slice_dims=(0,), start_index_map=(0,))

        def bcast_last(v):
            return lax.gather(
                v, idx15, gdn, (1,),
                mode=lax.GatherScatterMode.PROMISE_IN_BOUNDS)

        def start_in(ph):
            cur = ph % 2
            return [
                pltpu.async_copy(
                    pw_hbm.at[pl.ds(
                        (wid * _NQ + q) * w_q + ph * w_ph, w_ph)],
                    pbuf.at[cur, q], isems[cur])
                for q in range(_NQ)
            ]

        in_h = {0: start_in(0)}
        out_h = {}
        for ph in range(_NPH):
            cur = ph % 2
            if ph + 1 < _NPH:
                in_h[ph + 1] = start_in(ph + 1)
            for h in in_h.pop(ph):
                h.wait()
            if ph - 2 in out_h:
                for h in out_h.pop(ph - 2):
                    h.wait()

            def it(j, carry, cur=cur):
                new = []
                for q in range(_NQ):
                    pv = pbuf[cur, q, pl.ds(j * _L, _L)]
                    incl = plsc.cumsum(pv)
                    s = bcast_last(incl)
                    cbef = s * 0x01010100
                    b0 = incl & 0xFF
                    b1 = (incl >> 8) & 0xFF
                    b2 = (incl >> 16) & 0xFF
                    b3 = incl >> 24
                    c = carry[q]
                    o0 = c + b0
                    o1 = (c + ((cbef >> 8) & 0xFF)) + b1
                    o2 = (c + ((cbef >> 16) & 0xFF)) + b2
                    o3 = (c + (cbef >> 24)) + b3
                    zero = jnp.int32(0)
                    obuf[cur, q, pl.ds(j * 4 * _L, _L)] = jnp.minimum(
                        jnp.maximum(o0, zero), _MAX_VAL)
                    obuf[cur, q, pl.ds((j * 4 + 1) * _L, _L)] = jnp.minimum(
                        jnp.maximum(o1, zero), _MAX_VAL)
                    obuf[cur, q, pl.ds((j * 4 + 2) * _L, _L)] = jnp.minimum(
                        jnp.maximum(o2, zero), _MAX_VAL)
                    obuf[cur, q, pl.ds((j * 4 + 3) * _L, _L)] = jnp.minimum(
                        jnp.maximum(o3, zero), _MAX_VAL)
                    new.append(c + ((cbef >> 24) + (s >> 24)))
                return tuple(new)

            carry = list(lax.fori_loop(0, w_ph // _L, it, tuple(carry)))

            out_h[ph] = [
                pltpu.async_copy(
                    obuf.at[cur, q],
                    out_hbm.at[pl.ds(
                        wid * e_tile + q * e_q + ph * e_ph, e_ph)],
                    osems[cur])
                for q in range(_NQ)
            ]
        for hs in out_h.values():
            for h in hs:
                h.wait()

    def run(mask_i32):
        pw, sums = _pack_kernel(mask_i32)
        return _scan_kernel(pw, sums)

    return run


def kernel(mask_1d):
    n = mask_1d.shape[0]
    return _build(n)(mask_1d.astype(jnp.int32))


# speedup vs baseline: 1.2290x; 1.0590x over previous
import functools

import jax
import jax.numpy as jnp
from jax import lax
from jax.experimental import pallas as pl
from jax.experimental.pallas import tpu as pltpu
from jax.experimental.pallas import tpu_sc as plsc

_MAX_VAL = 2097151
_NC = 2
_NS = 16
_NW = _NC * _NS
_L = 16
_NSUB = 8


@functools.lru_cache(maxsize=None)
def _build(n):
    assert n % (_NW * 4 * _L * _NSUB) == 0, n
    e_tile = n // _NW
    e_sub = e_tile // _NSUB

    mesh = plsc.VectorSubcoreMesh(
        core_axis_name="c", subcore_axis_name="s",
        num_cores=_NC, num_subcores=_NS,
    )
    cparams = pltpu.CompilerParams(needs_layout_passes=False)

    @functools.partial(
        pl.kernel,
        out_type=jax.ShapeDtypeStruct((_NW, _L), jnp.int32),
        mesh=mesh,
        scratch_types=[
            pltpu.VMEM((2, e_sub), jnp.int32),
            pltpu.VMEM((_L,), jnp.int32),
            pltpu.SemaphoreType.DMA,
            pltpu.SemaphoreType.DMA,
        ],
        compiler_params=cparams,
    )
    def _sums_kernel(mask_hbm, out_hbm, buf, outv, isem0, isem1):
        wid = lax.axis_index("c") * _NS + lax.axis_index("s")
        base = wid * e_tile
        isems = (isem0, isem1)

        def start_in(sub):
            cur = sub % 2
            return pltpu.async_copy(
                mask_hbm.at[pl.ds(base + sub * e_sub, e_sub)],
                buf.at[cur], isems[cur])

        in_h = {0: start_in(0)}
        acc = jnp.zeros((_L,), jnp.int32)
        for sub in range(_NSUB):
            cur = sub % 2
            if sub + 1 < _NSUB:
                in_h[sub + 1] = start_in(sub + 1)
            in_h.pop(sub).wait()

            def it(i, acc, cur=cur):
                a = buf[cur, pl.ds(i * 4 * _L, _L)]
                b = buf[cur, pl.ds((i * 4 + 1) * _L, _L)]
                c = buf[cur, pl.ds((i * 4 + 2) * _L, _L)]
                d = buf[cur, pl.ds((i * 4 + 3) * _L, _L)]
                return acc + ((a + b) + (c + d))

            acc = lax.fori_loop(0, e_sub // (4 * _L), it, acc)

        outv[...] = acc
        pltpu.sync_copy(outv, out_hbm.at[wid])

    @functools.partial(
        pl.kernel,
        out_type=jax.ShapeDtypeStruct((n,), jnp.int32),
        mesh=mesh,
        scratch_types=[
            pltpu.VMEM((2, e_sub), jnp.int32),
            pltpu.VMEM((2, e_sub), jnp.int32),
            pltpu.VMEM((_NW, _L), jnp.int32),
            pltpu.SemaphoreType.DMA,
            pltpu.SemaphoreType.DMA,
            pltpu.SemaphoreType.DMA,
            pltpu.SemaphoreType.DMA,
        ],
        compiler_params=cparams,
    )
    def _scan_kernel(mask_hbm, sums_hbm, out_hbm, mbuf, obuf, sums_v,
                     isem0, isem1, osem0, osem1):
        wid = lax.axis_index("c") * _NS + lax.axis_index("s")
        base = wid * e_tile
        isems = (isem0, isem1)
        osems = (osem0, osem1)
        pltpu.sync_copy(sums_hbm, sums_v)

        def acc_row(wp, carryv):
            m = (wp < wid).astype(jnp.int32)
            return carryv + sums_v[wp] * m

        carry0 = lax.fori_loop(0, _NW, acc_row, jnp.zeros((_L,), jnp.int32))
        carry0 = jnp.sum(carry0) - 1

        def start_in(sub):
            cur = sub % 2
            return pltpu.async_copy(
                mask_hbm.at[pl.ds(base + sub * e_sub, e_sub)],
                mbuf.at[cur], isems[cur])

        in_h = {0: start_in(0)}
        out_h = {}
        carry = carry0
        for sub in range(_NSUB):
            cur = sub % 2
            if sub + 1 < _NSUB:
                in_h[sub + 1] = start_in(sub + 1)
            in_h.pop(sub).wait()
            if sub - 2 in out_h:
                out_h.pop(sub - 2).wait()

            def it(i, carry, cur=cur):
                v0 = mbuf[cur, pl.ds(i * 4 * _L, _L)]
                v1 = mbuf[cur, pl.ds((i * 4 + 1) * _L, _L)]
                v2 = mbuf[cur, pl.ds((i * 4 + 2) * _L, _L)]
                v3 = mbuf[cur, pl.ds((i * 4 + 3) * _L, _L)]
                packed = (v0 + (v1 << 8)) + ((v2 << 16) + (v3 << 24))
                incl = plsc.cumsum(packed)
                s = jnp.sum(packed)
                cbef = s * 0x01010100
                b0 = incl & 0xFF
                b1 = (incl >> 8) & 0xFF
                b2 = (incl >> 16) & 0xFF
                b3 = incl >> 24
                o0 = carry + b0
                o1 = (carry + ((cbef >> 8) & 0xFF)) + b1
                o2 = (carry + ((cbef >> 16) & 0xFF)) + b2
                o3 = (carry + (cbef >> 24)) + b3
                zero = jnp.int32(0)
                obuf[cur, pl.ds(i * 4 * _L, _L)] = jnp.minimum(
                    jnp.maximum(o0, zero), _MAX_VAL)
                obuf[cur, pl.ds((i * 4 + 1) * _L, _L)] = jnp.minimum(
                    jnp.maximum(o1, zero), _MAX_VAL)
                obuf[cur, pl.ds((i * 4 + 2) * _L, _L)] = jnp.minimum(
                    jnp.maximum(o2, zero), _MAX_VAL)
                obuf[cur, pl.ds((i * 4 + 3) * _L, _L)] = jnp.minimum(
                    jnp.maximum(o3, zero), _MAX_VAL)
                return carry + ((cbef >> 24) + (s >> 24))

            carry = lax.fori_loop(0, e_sub // (4 * _L), it, carry)
            out_h[sub] = pltpu.async_copy(
                obuf.at[cur], out_hbm.at[pl.ds(base + sub * e_sub, e_sub)],
                osems[cur])
        for h in out_h.values():
            h.wait()

    def run(mask_i32):
        sums = _sums_kernel(mask_i32)
        return _scan_kernel(mask_i32, sums)

    return run


def kernel(mask_1d):
    n = mask_1d.shape[0]
    return _build(n)(mask_1d.astype(jnp.int32))


# speedup vs baseline: 1.5010x; 1.2213x over previous
import functools

import jax
import jax.numpy as jnp
from jax import lax
from jax.experimental import pallas as pl
from jax.experimental.pallas import tpu as pltpu
from jax.experimental.pallas import tpu_sc as plsc

_MAX_VAL = 2097151
_NC = 2
_NS = 16
_NW = _NC * _NS
_L = 16
_NSUB = 8


@functools.lru_cache(maxsize=None)
def _build(n):
    assert n % (_NW * 4 * _L * _NSUB) == 0, n
    e_tile = n // _NW
    e_sub = e_tile // _NSUB

    mesh = plsc.VectorSubcoreMesh(
        core_axis_name="c", subcore_axis_name="s",
        num_cores=_NC, num_subcores=_NS,
    )
    cparams = pltpu.CompilerParams(needs_layout_passes=False)

    @functools.partial(
        pl.kernel,
        out_type=jax.ShapeDtypeStruct((_NW, _L), jnp.int32),
        mesh=mesh,
        scratch_types=[
            pltpu.VMEM((2, e_sub), jnp.int32),
            pltpu.VMEM((_L,), jnp.int32),
            pltpu.SemaphoreType.DMA,
            pltpu.SemaphoreType.DMA,
        ],
        compiler_params=cparams,
    )
    def _sums_kernel(mask_hbm, out_hbm, buf, outv, isem0, isem1):
        wid = lax.axis_index("c") * _NS + lax.axis_index("s")
        base = wid * e_tile
        isems = (isem0, isem1)

        def start_in(sub):
            cur = sub % 2
            return pltpu.async_copy(
                mask_hbm.at[pl.ds(base + sub * e_sub, e_sub)],
                buf.at[cur], isems[cur])

        in_h = {0: start_in(0)}
        acc = jnp.zeros((_L,), jnp.int32)
        for sub in range(_NSUB):
            cur = sub % 2
            if sub + 1 < _NSUB:
                in_h[sub + 1] = start_in(sub + 1)
            in_h.pop(sub).wait()

            def it(i, acc, cur=cur):
                a = buf[cur, pl.ds(i * 4 * _L, _L)]
                b = buf[cur, pl.ds((i * 4 + 1) * _L, _L)]
                c = buf[cur, pl.ds((i * 4 + 2) * _L, _L)]
                d = buf[cur, pl.ds((i * 4 + 3) * _L, _L)]
                return acc + ((a + b) + (c + d))

            acc = lax.fori_loop(0, e_sub // (4 * _L), it, acc)

        outv[...] = acc
        pltpu.sync_copy(outv, out_hbm.at[wid])

    @functools.partial(
        pl.kernel,
        out_type=jax.ShapeDtypeStruct((n,), jnp.int32),
        mesh=mesh,
        scratch_types=[
            pltpu.VMEM((e_sub,), jnp.int32),
            pltpu.VMEM((e_sub,), jnp.int32),
            pltpu.VMEM((_NW, _L), jnp.int32),
        ],
        compiler_params=cparams,
    )
    def _scan_kernel(mask_hbm, sums_hbm, out_hbm, mbuf, obuf, sums_v):
        wid = lax.axis_index("c") * _NS + lax.axis_index("s")
        base = wid * e_tile
        pltpu.sync_copy(sums_hbm, sums_v)

        def acc_row(wp, carryv):
            m = (wp < wid).astype(jnp.int32)
            return carryv + sums_v[wp] * m

        carry0 = lax.fori_loop(0, _NW, acc_row, jnp.zeros((_L,), jnp.int32))
        carry0 = jnp.sum(carry0) - 1

        def sub_body(sub, carry):
            pltpu.sync_copy(mask_hbm.at[pl.ds(base + sub * e_sub, e_sub)],
                            mbuf)

            def it(i, carry):
                v0 = mbuf[pl.ds(i * 4 * _L, _L)]
                v1 = mbuf[pl.ds((i * 4 + 1) * _L, _L)]
                v2 = mbuf[pl.ds((i * 4 + 2) * _L, _L)]
                v3 = mbuf[pl.ds((i * 4 + 3) * _L, _L)]
                packed = (v0 + (v1 << 8)) + ((v2 << 16) + (v3 << 24))
                incl = plsc.cumsum(packed)
                s = jnp.sum(packed)
                cbef = s * 0x01010100
                b0 = incl & 0xFF
                b1 = (incl >> 8) & 0xFF
                b2 = (incl >> 16) & 0xFF
                b3 = incl >> 24
                o0 = carry + b0
                o1 = (carry + ((cbef >> 8) & 0xFF)) + b1
                o2 = (carry + ((cbef >> 16) & 0xFF)) + b2
                o3 = (carry + (cbef >> 24)) + b3
                zero = jnp.int32(0)
                obuf[pl.ds(i * 4 * _L, _L)] = jnp.minimum(
                    jnp.maximum(o0, zero), _MAX_VAL)
                obuf[pl.ds((i * 4 + 1) * _L, _L)] = jnp.minimum(
                    jnp.maximum(o1, zero), _MAX_VAL)
                obuf[pl.ds((i * 4 + 2) * _L, _L)] = jnp.minimum(
                    jnp.maximum(o2, zero), _MAX_VAL)
                obuf[pl.ds((i * 4 + 3) * _L, _L)] = jnp.minimum(
                    jnp.maximum(o3, zero), _MAX_VAL)
                return carry + ((cbef >> 24) + (s >> 24))

            carry = lax.fori_loop(0, e_sub // (4 * _L), it, carry)
            pltpu.sync_copy(obuf, out_hbm.at[pl.ds(base + sub * e_sub, e_sub)])
            return carry

        lax.fori_loop(0, _NSUB, sub_body, carry0)

    def run(mask_i32):
        sums = _sums_kernel(mask_i32)
        return _scan_kernel(mask_i32, sums)

    return run


def kernel(mask_1d):
    n = mask_1d.shape[0]
    return _build(n)(mask_1d.astype(jnp.int32))
